# baseline (speedup 1.0000x reference)
import math

import jax
import jax.numpy as jnp
from jax.experimental import pallas as pl
from jax.experimental.pallas import tpu as pltpu


def _phase1_body(nk, half, bk, item):
    def body(x_ref, w_ref, ts_ref, freqs_ref, embW_ref, embb_ref, wt_ref,
             bin_ref, h_ref, ss_acc):
        k = pl.program_id(0)
        lim = item - k * bk
        cmask = jax.lax.broadcasted_iota(jnp.int32, (1, bk), 1) < lim
        rmask = jax.lax.broadcasted_iota(jnp.int32, (bk, 1), 0) < lim
        xb = jnp.where(cmask, x_ref[...], 0.0)
        wb = jnp.where(rmask, w_ref[...], 0.0)
        part = jnp.dot(xb, wb, preferred_element_type=jnp.float32)
        pss = jnp.sum(xb * xb, axis=1, keepdims=True)

        @pl.when(k == 0)
        def _():
            h_ref[...] = part
            ss_acc[...] = pss

        @pl.when(k > 0)
        def _():
            h_ref[...] = h_ref[...] + part
            ss_acc[...] = ss_acc[...] + pss

        @pl.when(k == nk - 1)
        def _():
            t = ts_ref[...].astype(jnp.float32)
            temp = t * freqs_ref[...]
            te = jnp.concatenate([jnp.cos(temp), jnp.sin(temp)], axis=-1)
            emb = jnp.dot(te, embW_ref[...],
                          preferred_element_type=jnp.float32) + embb_ref[...]
            contrib = jnp.dot(emb, wt_ref[...],
                              preferred_element_type=jnp.float32)
            norm = jnp.maximum(jnp.sqrt(ss_acc[...]), 1e-12)
            h_ref[...] = jnp.tanh(h_ref[...] / norm + contrib + bin_ref[...])

    return body


def _phase2_body(h_ref, w_ref, b_ref, o_ref):
    o_ref[...] = jnp.dot(h_ref[...], w_ref[...],
                         preferred_element_type=jnp.float32) + b_ref[...]


def kernel(x, timesteps, emb_W, emb_b, W_in, b_in, W_out, b_out):
    B, ITEM = x.shape
    HID = W_out.shape[0]
    TD = emb_W.shape[0]
    half = TD // 2

    bK = 3200
    NK = pl.cdiv(ITEM, bK)
    bN = 3200
    NN = pl.cdiv(ITEM, bN)

    ts2 = timesteps.reshape(B, 1)
    freqs = jnp.exp(-(math.log(10000.0) / half)
                    * jnp.arange(half, dtype=jnp.float32)).reshape(1, half)
    W_in_t = jax.lax.slice(W_in, (ITEM, 0), (ITEM + TD, HID))
    b_in2 = b_in.reshape(1, HID)
    emb_b2 = emb_b.reshape(1, TD)
    b_out2 = b_out.reshape(1, ITEM)

    h = pl.pallas_call(
        _phase1_body(NK, half, bK, ITEM),
        grid=(NK,),
        in_specs=[
            pl.BlockSpec((B, bK), lambda k: (0, k)),
            pl.BlockSpec((bK, HID), lambda k: (k, 0)),
            pl.BlockSpec((B, 1), lambda k: (0, 0)),
            pl.BlockSpec((1, half), lambda k: (0, 0)),
            pl.BlockSpec((TD, TD), lambda k: (0, 0)),
            pl.BlockSpec((1, TD), lambda k: (0, 0)),
            pl.BlockSpec((TD, HID), lambda k: (0, 0)),
            pl.BlockSpec((1, HID), lambda k: (0, 0)),
        ],
        out_specs=pl.BlockSpec((B, HID), lambda k: (0, 0)),
        out_shape=jax.ShapeDtypeStruct((B, HID), jnp.float32),
        scratch_shapes=[pltpu.VMEM((B, 1), jnp.float32)],
    )(x, W_in, ts2, freqs, emb_W, emb_b2, W_in_t, b_in2)

    out = pl.pallas_call(
        _phase2_body,
        grid=(NN,),
        in_specs=[
            pl.BlockSpec((B, HID), lambda n: (0, 0)),
            pl.BlockSpec((HID, bN), lambda n: (0, n)),
            pl.BlockSpec((1, bN), lambda n: (0, n)),
        ],
        out_specs=pl.BlockSpec((B, bN), lambda n: (0, n)),
        out_shape=jax.ShapeDtypeStruct((B, ITEM), jnp.float32),
    )(h, W_out, b_out2)

    return out
